# baseline (speedup 1.0000x reference)
import functools

import jax
import jax.numpy as jnp
import numpy as np
from jax import lax
from jax.experimental import pallas as pl
from jax.experimental.pallas import tpu as pltpu
from jax.experimental.pallas import tpu_sc as plsc

_LS = 0.2
_C = 21
_AVAL = np.float32(np.float32(0.0) * np.float32(1.0 - _LS) + np.float32(_LS / _C))
_BVAL = np.float32(np.float32(1.0) * np.float32(1.0 - _LS) + np.float32(_LS / _C))

_B = 16384
_NBIN2 = 2 * _B
_P = 4 * 384 * 384
_PW = _P // 2
_NPART = 3
_PART = _PW // _NPART
_CHUNK = 16384
_NCHUNK = _PART // _CHUNK
_NTASK = _C * _NPART
_NWORKER = 32


def _stage1_body(x_ref, t_ref, out_ref):
    xb = x_ref[0]
    tgt = t_ref[0]
    m = jnp.max(xb, axis=0)
    ex = jnp.exp(xb - m[None])
    p = ex / jnp.sum(ex, axis=0)[None]
    cls = lax.broadcasted_iota(jnp.int32, xb.shape, 0)
    isfg = cls == tgt[None]
    fgv = jnp.where(isfg, _BVAL, _AVAL)
    err = jnp.abs(fgv - p)
    b = jnp.minimum((err * np.float32(_B)).astype(jnp.int32), _B - 1)
    enc = (b | jnp.where(isfg, _B, 0)).astype(jnp.uint32)
    packed = enc[:, :24, :] | (enc[:, 24:, :] << jnp.uint32(16))
    out_ref[:, 0] = packed


def _stage1(x, target):
    rows = 48
    grid = (4, 384 // rows)
    return pl.pallas_call(
        _stage1_body,
        grid=grid,
        in_specs=[
            pl.BlockSpec((1, _C, rows, 384), lambda b, r: (b, 0, r, 0)),
            pl.BlockSpec((1, rows, 384), lambda b, r: (b, r, 0)),
        ],
        out_specs=pl.BlockSpec((_C, 1, rows // 2, 384), lambda b, r: (0, b, r, 0)),
        out_shape=jax.ShapeDtypeStruct((_C, 4, 192, 384), jnp.uint32),
    )(x, target)


def _sc_hist_body(packed_hbm, out_hbm, chunk0_v, chunk1_v, cnta_v, cntb_v, sem0, sem1):
    wid = lax.axis_index("s") * 2 + lax.axis_index("c")
    bufs = (chunk0_v, chunk1_v)
    sems = (sem0, sem1)
    for tt in range(2):
        t = wid + _NWORKER * tt

        @pl.when(t < _NTASK)
        def _():
            cls = t // _NPART
            base = (t % _NPART) * _PART

            def zbody(i, carry):
                z = jnp.zeros((16,), jnp.float32)
                cnta_v[pl.ds(i * 16, 16)] = z
                cntb_v[pl.ds(i * 16, 16)] = z
                return carry

            lax.fori_loop(0, _NBIN2 // 16, zbody, 0, unroll=8)

            def start(k, buf, sem):
                return pltpu.async_copy(
                    packed_hbm.at[cls, pl.ds(base + k * _CHUNK, _CHUNK)], buf, sem
                )

            def consume(buf):
                ones = jnp.ones((16,), jnp.float32)

                def vbody(i, c2):
                    u = buf[pl.ds(i * 16, 16)]
                    ilo = (u & jnp.uint32(0x7FFF)).astype(jnp.int32)
                    ihi = (u >> jnp.uint32(16)).astype(jnp.int32)
                    plsc.addupdate_scatter(cnta_v, [ilo], ones)
                    plsc.addupdate_scatter(cntb_v, [ihi], ones)
                    return c2

                lax.fori_loop(0, _CHUNK // 16, vbody, 0, unroll=16)

            pending = start(0, bufs[0], sems[0])
            for k in range(_NCHUNK):
                nxt = None
                if k + 1 < _NCHUNK:
                    nxt = start(k + 1, bufs[(k + 1) % 2], sems[(k + 1) % 2])
                pending.wait()
                consume(bufs[k % 2])
                pending = nxt
            pltpu.sync_copy(cnta_v, out_hbm.at[t, 0])
            pltpu.sync_copy(cntb_v, out_hbm.at[t, 1])


def _sc_histograms(packed_flat):
    mesh = plsc.VectorSubcoreMesh(core_axis_name="c", subcore_axis_name="s")
    f = functools.partial(
        pl.kernel,
        mesh=mesh,
        compiler_params=pltpu.CompilerParams(needs_layout_passes=False),
        out_type=jax.ShapeDtypeStruct((_NTASK, 2, _NBIN2), jnp.float32),
        scratch_types=[
            pltpu.VMEM((_CHUNK,), jnp.uint32),
            pltpu.VMEM((_CHUNK,), jnp.uint32),
            pltpu.VMEM((_NBIN2,), jnp.float32),
            pltpu.VMEM((_NBIN2,), jnp.float32),
            pltpu.SemaphoreType.DMA,
            pltpu.SemaphoreType.DMA,
        ],
    )(_sc_hist_body)
    return f(packed_flat)


def _stage3_body(h_ref, out_ref):
    c = pl.program_id(0)
    hh = h_ref[0]
    hr = (hh[0, 0] + hh[0, 1]) + (hh[1, 0] + hh[1, 1]) + (hh[2, 0] + hh[2, 1])
    nbg = hr[0]
    nfg = hr[1]
    nall = nbg + nfg

    f32 = jnp.float32
    ii = lax.broadcasted_iota(jnp.int32, (128, 128), 0)
    jj = lax.broadcasted_iota(jnp.int32, (128, 128), 1)
    m_incl = (ii <= jj).astype(f32)
    ones128 = jnp.ones((128, 128), f32)
    a_geq = (jj >= ii).astype(f32)
    mids = ((ii * 128 + jj).astype(f32) + f32(0.5)) * f32(1.0 / _B)
    sfg = nfg * mids
    sbg = nbg * mids

    def mm(a, b):
        return lax.dot_general(
            a, b, (((1,), (0,)), ((), ())),
            preferred_element_type=f32, precision=lax.Precision.HIGHEST,
        )

    wf = mm(nfg, m_incl)
    wa = mm(nall, m_incl)
    sf = mm(nfg, ones128)
    sa = mm(nall, ones128)
    ff_above = mm(a_geq, sf) - wf
    ta_above = mm(a_geq, sa) - wa
    nf_b = mm(ones128, sf)
    na_b = mm(ones128, sa)
    g = _AVAL * na_b + (_BVAL - _AVAL) * nf_b

    def jac(n, k):
        s = k * _BVAL + (n - k) * _AVAL
        return 1.0 - (g - s) / (g + n - s)

    j0 = jac(ta_above, ff_above)
    j1 = jac(ta_above + nfg, ff_above + nfg)
    j2 = jac(ta_above + nall, ff_above + nfg)
    contrib = sfg * (j1 - j0) / jnp.maximum(nfg, 1.0) + sbg * (j2 - j1) / jnp.maximum(
        nbg, 1.0
    )
    cl = jnp.sum(contrib) / np.float32(_C)

    @pl.when(c == 0)
    def _():
        out_ref[0, 0] = jnp.float32(0.0)

    out_ref[0, 0] += cl


def _stage3(hist):
    h = hist.reshape(_C, _NPART, 2, 2, 128, 128)
    out = pl.pallas_call(
        _stage3_body,
        grid=(_C,),
        in_specs=[pl.BlockSpec((1, _NPART, 2, 2, 128, 128), lambda c: (c, 0, 0, 0, 0, 0))],
        out_specs=pl.BlockSpec(memory_space=pltpu.SMEM),
        out_shape=jax.ShapeDtypeStruct((1, 1), jnp.float32),
    )(h)
    return out


def kernel(x, target):
    packed = _stage1(x, target.astype(jnp.int32))
    hist = _sc_histograms(packed.reshape(_C, _PW))
    return _stage3(hist).reshape(())


# speedup vs baseline: 1.3172x; 1.3172x over previous
import functools

import jax
import jax.numpy as jnp
import numpy as np
from jax import lax
from jax.experimental import pallas as pl
from jax.experimental.pallas import tpu as pltpu
from jax.experimental.pallas import tpu_sc as plsc

_LS = 0.2
_C = 21
_AVAL = np.float32(np.float32(0.0) * np.float32(1.0 - _LS) + np.float32(_LS / _C))
_BVAL = np.float32(np.float32(1.0) * np.float32(1.0 - _LS) + np.float32(_LS / _C))

_B = 16384
_NBIN2 = 2 * _B
_P = 4 * 384 * 384
_PW = _P // 2
_NPART = 3
_PART = _PW // _NPART
_CHUNK = 16384
_NCHUNK = _PART // _CHUNK
_NTASK = _C * _NPART
_NWORKER = 32


def _stage1_body(x_ref, t_ref, out_ref):
    xb = x_ref[0]
    tgt = t_ref[0]
    m = jnp.max(xb, axis=0)
    ex = jnp.exp(xb - m[None])
    p = ex / jnp.sum(ex, axis=0)[None]
    cls = lax.broadcasted_iota(jnp.int32, xb.shape, 0)
    isfg = cls == tgt[None]
    fgv = jnp.where(isfg, _BVAL, _AVAL)
    err = jnp.abs(fgv - p)
    b = jnp.minimum((err * np.float32(_B)).astype(jnp.int32), _B - 1)
    enc = (b | jnp.where(isfg, _B, 0)).astype(jnp.uint32)
    packed = enc[:, :24, :] | (enc[:, 24:, :] << jnp.uint32(16))
    out_ref[...] = packed.reshape(_C, 24 * 384)


def _stage1(x, target):
    rows = 48
    grid = (4, 384 // rows)
    return pl.pallas_call(
        _stage1_body,
        grid=grid,
        in_specs=[
            pl.BlockSpec((1, _C, rows, 384), lambda b, r: (b, 0, r, 0)),
            pl.BlockSpec((1, rows, 384), lambda b, r: (b, r, 0)),
        ],
        out_specs=pl.BlockSpec((_C, 24 * 384), lambda b, r: (0, b * 8 + r)),
        out_shape=jax.ShapeDtypeStruct((_C, _PW), jnp.uint32),
    )(x, target)


def _sc_hist_body(packed_hbm, out_hbm, chunk0_v, chunk1_v, cnta_v, cntb_v, sem0, sem1):
    wid = lax.axis_index("s") * 2 + lax.axis_index("c")
    bufs = (chunk0_v, chunk1_v)
    sems = (sem0, sem1)
    for tt in range(2):
        t = wid + _NWORKER * tt

        @pl.when(t < _NTASK)
        def _():
            cls = t // _NPART
            base = (t % _NPART) * _PART

            def zbody(r, carry):
                z = jnp.zeros((16,), jnp.float32)
                for h in range(2):
                    for k in range(8):
                        cnta_v[h, r, pl.ds(k * 16, 16)] = z
                        cntb_v[h, r, pl.ds(k * 16, 16)] = z
                return carry

            lax.fori_loop(0, 128, zbody, 0, unroll=2)

            def start(k, buf, sem):
                return pltpu.async_copy(
                    packed_hbm.at[cls, pl.ds(base + k * _CHUNK, _CHUNK)], buf, sem
                )

            def consume(buf):
                ones = jnp.ones((16,), jnp.float32)

                def split3(v):
                    return [
                        (v >> jnp.uint32(14)).astype(jnp.int32),
                        ((v >> jnp.uint32(7)) & jnp.uint32(127)).astype(jnp.int32),
                        (v & jnp.uint32(127)).astype(jnp.int32),
                    ]

                def vbody(i, c2):
                    u = buf[pl.ds(i * 16, 16)]
                    ilo = u & jnp.uint32(0x7FFF)
                    ihi = u >> jnp.uint32(16)
                    plsc.addupdate_scatter(cnta_v, split3(ilo), ones)
                    plsc.addupdate_scatter(cntb_v, split3(ihi), ones)
                    return c2

                lax.fori_loop(0, _CHUNK // 16, vbody, 0, unroll=16)

            pending = start(0, bufs[0], sems[0])
            for k in range(_NCHUNK):
                nxt = None
                if k + 1 < _NCHUNK:
                    nxt = start(k + 1, bufs[(k + 1) % 2], sems[(k + 1) % 2])
                pending.wait()
                consume(bufs[k % 2])
                pending = nxt
            cls2 = t // _NPART
            part = t % _NPART
            pltpu.sync_copy(cnta_v, out_hbm.at[cls2, part, 0])
            pltpu.sync_copy(cntb_v, out_hbm.at[cls2, part, 1])


def _sc_histograms(packed_flat):
    mesh = plsc.VectorSubcoreMesh(core_axis_name="c", subcore_axis_name="s")
    f = functools.partial(
        pl.kernel,
        mesh=mesh,
        compiler_params=pltpu.CompilerParams(needs_layout_passes=False),
        out_type=jax.ShapeDtypeStruct((_C, _NPART, 2, 2, 128, 128), jnp.float32),
        scratch_types=[
            pltpu.VMEM((_CHUNK,), jnp.uint32),
            pltpu.VMEM((_CHUNK,), jnp.uint32),
            pltpu.VMEM((2, 128, 128), jnp.float32),
            pltpu.VMEM((2, 128, 128), jnp.float32),
            pltpu.SemaphoreType.DMA,
            pltpu.SemaphoreType.DMA,
        ],
    )(_sc_hist_body)
    return f(packed_flat)


def _stage3_body(h_ref, out_ref):
    c = pl.program_id(0)
    hh = h_ref[0]
    hr = (hh[0, 0] + hh[0, 1]) + (hh[1, 0] + hh[1, 1]) + (hh[2, 0] + hh[2, 1])
    nbg = hr[0]
    nfg = hr[1]
    nall = nbg + nfg

    f32 = jnp.float32
    ii = lax.broadcasted_iota(jnp.int32, (128, 128), 0)
    jj = lax.broadcasted_iota(jnp.int32, (128, 128), 1)
    m_incl = (ii <= jj).astype(f32)
    ones128 = jnp.ones((128, 128), f32)
    a_geq = (jj >= ii).astype(f32)
    mids = ((ii * 128 + jj).astype(f32) + f32(0.5)) * f32(1.0 / _B)
    sfg = nfg * mids
    sbg = nbg * mids

    def mm(a, b):
        return lax.dot_general(
            a, b, (((1,), (0,)), ((), ())),
            preferred_element_type=f32, precision=lax.Precision.HIGHEST,
        )

    wf = mm(nfg, m_incl)
    wa = mm(nall, m_incl)
    sf = mm(nfg, ones128)
    sa = mm(nall, ones128)
    ff_above = mm(a_geq, sf) - wf
    ta_above = mm(a_geq, sa) - wa
    nf_b = mm(ones128, sf)
    na_b = mm(ones128, sa)
    g = _AVAL * na_b + (_BVAL - _AVAL) * nf_b

    def jac(n, k):
        s = k * _BVAL + (n - k) * _AVAL
        return 1.0 - (g - s) / (g + n - s)

    j0 = jac(ta_above, ff_above)
    j1 = jac(ta_above + nfg, ff_above + nfg)
    j2 = jac(ta_above + nall, ff_above + nfg)
    contrib = sfg * (j1 - j0) / jnp.maximum(nfg, 1.0) + sbg * (j2 - j1) / jnp.maximum(
        nbg, 1.0
    )
    cl = jnp.sum(contrib) / np.float32(_C)

    @pl.when(c == 0)
    def _():
        out_ref[0, 0] = jnp.float32(0.0)

    out_ref[0, 0] += cl


def _stage3(hist):
    h = hist
    out = pl.pallas_call(
        _stage3_body,
        grid=(_C,),
        in_specs=[pl.BlockSpec((1, _NPART, 2, 2, 128, 128), lambda c: (c, 0, 0, 0, 0, 0))],
        out_specs=pl.BlockSpec(memory_space=pltpu.SMEM),
        out_shape=jax.ShapeDtypeStruct((1, 1), jnp.float32),
    )(h)
    return out


def kernel(x, target):
    packed = _stage1(x, target.astype(jnp.int32))
    hist = _sc_histograms(packed)
    return _stage3(hist).reshape(())


# speedup vs baseline: 1.3676x; 1.0383x over previous
import functools

import jax
import jax.numpy as jnp
import numpy as np
from jax import lax
from jax.experimental import pallas as pl
from jax.experimental.pallas import tpu as pltpu
from jax.experimental.pallas import tpu_sc as plsc

_LS = 0.2
_C = 21
_AVAL = np.float32(np.float32(0.0) * np.float32(1.0 - _LS) + np.float32(_LS / _C))
_BVAL = np.float32(np.float32(1.0) * np.float32(1.0 - _LS) + np.float32(_LS / _C))

_B = 16384
_NBIN2 = 2 * _B
_P = 4 * 384 * 384
_PW = _P // 2
_NPART = 3
_PART = _PW // _NPART
_CHUNK = 16384
_NCHUNK = _PART // _CHUNK
_NTASK = _C * _NPART
_NWORKER = 32


def _stage1_body(x_ref, t_ref, out_ref):
    xb = x_ref[0]
    tgt = t_ref[0]
    m = jnp.max(xb, axis=0)
    ex = jnp.exp(xb - m[None])
    p = ex / jnp.sum(ex, axis=0)[None]
    cls = lax.broadcasted_iota(jnp.int32, xb.shape, 0)
    isfg = cls == tgt[None]
    fgv = jnp.where(isfg, _BVAL, _AVAL)
    err = jnp.abs(fgv - p)
    b = jnp.minimum((err * np.float32(_B)).astype(jnp.int32), _B - 1)
    enc = (b | jnp.where(isfg, _B, 0)).astype(jnp.uint32)
    hrows = xb.shape[1] // 2
    packed = enc[:, :hrows, :] | (enc[:, hrows:, :] << jnp.uint32(16))
    out_ref[...] = packed.reshape(_C, hrows * 384)


def _stage1(x, target):
    rows = 96
    grid = (4, 384 // rows)
    return pl.pallas_call(
        _stage1_body,
        grid=grid,
        in_specs=[
            pl.BlockSpec((1, _C, rows, 384), lambda b, r: (b, 0, r, 0)),
            pl.BlockSpec((1, rows, 384), lambda b, r: (b, r, 0)),
        ],
        out_specs=pl.BlockSpec((_C, 48 * 384), lambda b, r: (0, b * 4 + r)),
        out_shape=jax.ShapeDtypeStruct((_C, _PW), jnp.uint32),
    )(x, target)


def _sc_hist_body(packed_hbm, out_hbm, chunk0_v, chunk1_v, cnta_v, cntb_v, sem0, sem1):
    wid = lax.axis_index("s") * 2 + lax.axis_index("c")
    bufs = (chunk0_v, chunk1_v)
    sems = (sem0, sem1)
    for tt in range(2):
        t = wid + _NWORKER * tt

        @pl.when(t < _NTASK)
        def _():
            cls = t // _NPART
            base = (t % _NPART) * _PART

            def zbody(r, carry):
                z = jnp.zeros((16,), jnp.float32)
                for h in range(2):
                    for k in range(8):
                        cnta_v[h, r, pl.ds(k * 16, 16)] = z
                        cntb_v[h, r, pl.ds(k * 16, 16)] = z
                return carry

            lax.fori_loop(0, 128, zbody, 0, unroll=2)

            def start(k, buf, sem):
                return pltpu.async_copy(
                    packed_hbm.at[cls, pl.ds(base + k * _CHUNK, _CHUNK)], buf, sem
                )

            def consume(buf):
                ones = jnp.ones((16,), jnp.float32)

                def split3(v):
                    return [
                        (v >> jnp.uint32(14)).astype(jnp.int32),
                        ((v >> jnp.uint32(7)) & jnp.uint32(127)).astype(jnp.int32),
                        (v & jnp.uint32(127)).astype(jnp.int32),
                    ]

                def vbody(i, c2):
                    u = buf[pl.ds(i * 16, 16)]
                    ilo = u & jnp.uint32(0x7FFF)
                    ihi = u >> jnp.uint32(16)
                    plsc.addupdate_scatter(cnta_v, split3(ilo), ones)
                    plsc.addupdate_scatter(cntb_v, split3(ihi), ones)
                    return c2

                lax.fori_loop(0, _CHUNK // 16, vbody, 0, unroll=16)

            pending = start(0, bufs[0], sems[0])
            for k in range(_NCHUNK):
                nxt = None
                if k + 1 < _NCHUNK:
                    nxt = start(k + 1, bufs[(k + 1) % 2], sems[(k + 1) % 2])
                pending.wait()
                consume(bufs[k % 2])
                pending = nxt
            cls2 = t // _NPART
            part = t % _NPART
            pltpu.sync_copy(cnta_v, out_hbm.at[cls2, part, 0])
            pltpu.sync_copy(cntb_v, out_hbm.at[cls2, part, 1])


def _sc_histograms(packed_flat):
    mesh = plsc.VectorSubcoreMesh(core_axis_name="c", subcore_axis_name="s")
    f = functools.partial(
        pl.kernel,
        mesh=mesh,
        compiler_params=pltpu.CompilerParams(needs_layout_passes=False),
        out_type=jax.ShapeDtypeStruct((_C, _NPART, 2, 2, 128, 128), jnp.float32),
        scratch_types=[
            pltpu.VMEM((_CHUNK,), jnp.uint32),
            pltpu.VMEM((_CHUNK,), jnp.uint32),
            pltpu.VMEM((2, 128, 128), jnp.float32),
            pltpu.VMEM((2, 128, 128), jnp.float32),
            pltpu.SemaphoreType.DMA,
            pltpu.SemaphoreType.DMA,
        ],
    )(_sc_hist_body)
    return f(packed_flat)


def _stage3_body(h_ref, out_ref):
    c = pl.program_id(0)
    hh = h_ref[0]
    hr = (hh[0, 0] + hh[0, 1]) + (hh[1, 0] + hh[1, 1]) + (hh[2, 0] + hh[2, 1])
    nbg = hr[0]
    nfg = hr[1]
    nall = nbg + nfg

    f32 = jnp.float32
    ii = lax.broadcasted_iota(jnp.int32, (128, 128), 0)
    jj = lax.broadcasted_iota(jnp.int32, (128, 128), 1)
    m_incl = (ii <= jj).astype(f32)
    ones128 = jnp.ones((128, 128), f32)
    a_geq = (jj >= ii).astype(f32)
    mids = ((ii * 128 + jj).astype(f32) + f32(0.5)) * f32(1.0 / _B)
    sfg = nfg * mids
    sbg = nbg * mids

    def mm(a, b):
        return lax.dot_general(
            a, b, (((1,), (0,)), ((), ())),
            preferred_element_type=f32, precision=lax.Precision.HIGHEST,
        )

    wf = mm(nfg, m_incl)
    wa = mm(nall, m_incl)
    sf = mm(nfg, ones128)
    sa = mm(nall, ones128)
    ff_above = mm(a_geq, sf) - wf
    ta_above = mm(a_geq, sa) - wa
    nf_b = mm(ones128, sf)
    na_b = mm(ones128, sa)
    g = _AVAL * na_b + (_BVAL - _AVAL) * nf_b

    def jac(n, k):
        s = k * _BVAL + (n - k) * _AVAL
        return 1.0 - (g - s) / (g + n - s)

    j0 = jac(ta_above, ff_above)
    j1 = jac(ta_above + nfg, ff_above + nfg)
    j2 = jac(ta_above + nall, ff_above + nfg)
    contrib = sfg * (j1 - j0) / jnp.maximum(nfg, 1.0) + sbg * (j2 - j1) / jnp.maximum(
        nbg, 1.0
    )
    cl = jnp.sum(contrib) / np.float32(_C)

    @pl.when(c == 0)
    def _():
        out_ref[0, 0] = jnp.float32(0.0)

    out_ref[0, 0] += cl


def _stage3(hist):
    h = hist
    out = pl.pallas_call(
        _stage3_body,
        grid=(_C,),
        in_specs=[pl.BlockSpec((1, _NPART, 2, 2, 128, 128), lambda c: (c, 0, 0, 0, 0, 0))],
        out_specs=pl.BlockSpec(memory_space=pltpu.SMEM),
        out_shape=jax.ShapeDtypeStruct((1, 1), jnp.float32),
    )(h)
    return out


def kernel(x, target):
    packed = _stage1(x, target.astype(jnp.int32))
    hist = _sc_histograms(packed)
    return _stage3(hist).reshape(())
